# baseline (speedup 1.0000x reference)
import functools
import math

import jax
import jax.numpy as jnp
from jax import lax
from jax.experimental import pallas as pl
from jax.experimental.pallas import tpu as pltpu
from jax.experimental.pallas import tpu_sc as plsc

N = 10000
E = 160000
EPAD = 163840
NCORE = 2
NSUB = 16
NW = NCORE * NSUB
BN = 1000
EPS = 1e-16

_SC_PARAMS = pltpu.CompilerParams(
    use_tc_tiling_on_sc=False, needs_layout_passes=False)


def _mesh():
    return plsc.VectorSubcoreMesh(core_axis_name="c", subcore_axis_name="s")


def _make_phase_a(D, H, C, HG):
    DH = D // HG
    HH = H // HG
    EPT = EPAD // NW
    BA = 64
    NCH = EPT // BA
    inv = 1.0 / math.sqrt(C)

    @functools.partial(
        pl.kernel,
        mesh=_mesh(),
        out_type=jax.ShapeDtypeStruct((H, EPAD), jnp.float32),
        scratch_types=[
            pltpu.VMEM((2, BA), jnp.int32),
            pltpu.VMEM((2, BA), jnp.int32),
            pltpu.VMEM((BA, DH), jnp.float32),
            pltpu.VMEM((BA, DH), jnp.float32),
            pltpu.VMEM((BA, DH), jnp.float32),
            pltpu.VMEM((BA, DH), jnp.float32),
            pltpu.VMEM((HH, BA), jnp.float32),
            pltpu.VMEM((HH, BA), jnp.float32),
            pltpu.SemaphoreType.DMA,
            pltpu.SemaphoreType.DMA,
            pltpu.SemaphoreType.DMA,
            pltpu.SemaphoreType.DMA,
        ],
        compiler_params=_SC_PARAMS,
        name=f"edge_attn_w_D{D}",
    )
    def ka(*refs):
        q_tabs = refs[0:HG]
        k_tabs = refs[HG:2 * HG]
        ei_hbm = refs[2 * HG]
        wt_hbm = refs[2 * HG + 1]
        (ebufA, ebufB, qrA, qrB, krA, krB, wbufA, wbufB, semA, semB,
         wsemA, wsemB) = refs[2 * HG + 2:]
        cid = lax.axis_index("c")
        sid = lax.axis_index("s")
        wid = cid * NSUB + sid
        base = wid * EPT
        lane = lax.broadcasted_iota(jnp.int32, (16,), 0)

        def start_gathers(q_hbm, k_hbm, ebuf, qr, kr, sem, off):
            pltpu.sync_copy(ei_hbm.at[:, pl.ds(off, BA)], ebuf)
            ck = pltpu.async_copy(k_hbm.at[ebuf.at[0]], kr, sem)
            cq = pltpu.async_copy(q_hbm.at[ebuf.at[1]], qr, sem)
            return ck, cq

        for p in range(HG):
            q_hbm = q_tabs[p]
            k_hbm = k_tabs[p]

            def issue(ebuf, qr, kr, sem, ci):
                off = pl.multiple_of(base + ci * BA, 8)
                return start_gathers(q_hbm, k_hbm, ebuf, qr, kr, sem, off)

            perms = [lane ^ s for s in (1, 2, 4, 8)]

            def hsum(v):
                for pm in perms:
                    v = v + v.at[pm].get(mode="promise_in_bounds")
                return v

            def process(ebuf, qr, kr, sem, ci, wbuf, wsem):
                pltpu.make_async_copy(k_hbm.at[ebuf.at[0]], kr, sem).wait()
                pltpu.make_async_copy(q_hbm.at[ebuf.at[1]], qr, sem).wait()
                off = pl.multiple_of(base + ci * BA, 8)

                @pl.when(ci >= 2)
                def _():
                    pltpu.make_async_copy(
                        wbuf,
                        wt_hbm.at[pl.ds(p * HH, HH), pl.ds(off, BA)],
                        wsem).wait()

                def group(g, carry):
                    valid = (off + g * 16 + lane) < E
                    wvecs = [jnp.zeros((16,), jnp.float32)] * HH
                    for j in range(16):
                        e = g * 16 + j
                        for h in range(HH):
                            acc = None
                            for t in range(C // 16):
                                sl = pl.ds(h * C + t * 16, 16)
                                prod = qr[e, sl] * kr[e, sl]
                                acc = prod if acc is None else acc + prod
                            s = hsum(acc)
                            wvecs[h] = jnp.where(lane == j, s, wvecs[h])
                    for h in range(HH):
                        w = jnp.exp(wvecs[h] * inv)
                        w = jnp.where(valid, w, 0.0)
                        wbuf[h, pl.ds(g * 16, 16)] = w
                    return carry

                lax.fori_loop(0, BA // 16, group, 0)
                pltpu.async_copy(
                    wbuf,
                    wt_hbm.at[pl.ds(p * HH, HH), pl.ds(off, BA)], wsem)

            issue(ebufA, qrA, krA, semA, 0)
            issue(ebufB, qrB, krB, semB, 1)

            def pair(kk, carry):
                i0 = kk * 2
                process(ebufA, qrA, krA, semA, i0, wbufA, wsemA)

                @pl.when(kk < NCH // 2 - 1)
                def _():
                    issue(ebufA, qrA, krA, semA, i0 + 2)

                process(ebufB, qrB, krB, semB, i0 + 1, wbufB, wsemB)

                @pl.when(kk < NCH // 2 - 1)
                def _():
                    issue(ebufB, qrB, krB, semB, i0 + 3)

                return carry

            lax.fori_loop(0, NCH // 2, pair, 0)
            pltpu.make_async_copy(
                wbufA, wt_hbm.at[pl.ds(p * HH, HH), pl.ds(base, BA)],
                wsemA).wait()
            pltpu.make_async_copy(
                wbufB, wt_hbm.at[pl.ds(p * HH, HH), pl.ds(base, BA)],
                wsemB).wait()

    return ka


def _make_phase_b(D, H, C, NCHUNK):
    CW = 2 * C
    CWA = CW + 16
    JH = C // 16
    NPASS = NCHUNK // NCORE
    EPT2 = EPAD // NSUB
    BB = 128
    NCB = EPT2 // BB
    ROWS_PT = N // NSUB
    ZR = 25

    @functools.partial(
        pl.kernel,
        mesh=_mesh(),
        out_type=jax.ShapeDtypeStruct((NCHUNK * N, CWA), jnp.float32),
        scratch_types=[
            pltpu.VMEM((2, BB), jnp.int32),
            pltpu.VMEM((2, BB), jnp.int32),
            pltpu.VMEM((BB,), jnp.int32),
            pltpu.VMEM((BB,), jnp.int32),
            pltpu.VMEM((2, BB), jnp.float32),
            pltpu.VMEM((2, BB), jnp.float32),
            pltpu.VMEM((BB, CWA), jnp.float32),
            pltpu.VMEM((BB, CWA), jnp.float32),
            pltpu.VMEM_SHARED((N, CWA), jnp.float32),
            pltpu.SemaphoreType.DMA,
            pltpu.SemaphoreType.DMA,
            pltpu.SemaphoreType.DMA,
            pltpu.SemaphoreType.DMA,
        ],
        compiler_params=_SC_PARAMS,
        name=f"edge_scatter_D{D}",
    )
    def kb(vaug_hbm, ei_hbm, wt_hbm, out_hbm, ebufA, ebufB, gidxA, gidxB,
           wbufA, wbufB, rowsA, rowsB, numsh, semA, semB, ssemA, ssemB):
        cid = lax.axis_index("c")
        sid = lax.axis_index("s")
        lane = lax.broadcasted_iota(jnp.int32, (16,), 0)
        m0 = jnp.where(lane == 0, 1.0, 0.0)
        m1 = jnp.where(lane == 1, 1.0, 0.0)
        zero16 = jnp.zeros((16,), jnp.float32)
        row0 = sid * ROWS_PT

        for p in range(NPASS):
            q = p * NCORE + cid
            qN = q * N
            h0 = q * 2
            for zr in range(ZR):
                for j in range(CWA // 16):
                    rowsA[zr, pl.ds(j * 16, 16)] = zero16
            for zi in range(ROWS_PT // ZR):
                pltpu.sync_copy(rowsA.at[pl.ds(0, ZR)],
                                numsh.at[pl.ds(row0 + zi * ZR, ZR)])
            plsc.subcore_barrier()

            def issue(ebuf, gidx, wbuf, rows, sem, ci):
                off = pl.multiple_of(sid * EPT2 + ci * BB, 8)
                pltpu.sync_copy(ei_hbm.at[:, pl.ds(off, BB)], ebuf)
                pltpu.sync_copy(
                    wt_hbm.at[pl.ds(h0, 2), pl.ds(off, BB)], wbuf)
                for j in range(BB // 16):
                    sl = pl.ds(j * 16, 16)
                    gidx[sl] = ebuf[0, sl] + qN
                pltpu.async_copy(vaug_hbm.at[gidx], rows, sem)

            def process(ebuf, gidx, wbuf, rows, sem):
                pltpu.make_async_copy(vaug_hbm.at[gidx], rows, sem).wait()

                def group(g, c2):
                    w0vec = wbuf[0, pl.ds(g * 16, 16)]
                    w1vec = wbuf[1, pl.ds(g * 16, 16)]
                    for j in range(16):
                        e = g * 16 + j
                        jsplat = jnp.full((16,), j, jnp.int32)
                        w0 = w0vec.at[jsplat].get(mode="promise_in_bounds")
                        w1 = w1vec.at[jsplat].get(mode="promise_in_bounds")
                        s8 = w0 * m0 + w1 * m1
                        for t in range(JH):
                            sl = pl.ds(t * 16, 16)
                            rows[e, sl] = rows[e, sl] * w0
                        for t in range(JH):
                            sl = pl.ds(C + t * 16, 16)
                            rows[e, sl] = rows[e, sl] * w1
                        sl = pl.ds(CW, 16)
                        rows[e, sl] = rows[e, sl] * s8
                    return c2

                lax.fori_loop(0, BB // 16, group, 0)

            def scat(ebuf, rows, ssem):
                pltpu.async_copy(rows, numsh.at[ebuf.at[1]], ssem, add=True)

            def drain(ebuf, rows, ssem):
                pltpu.make_async_copy(rows, numsh.at[ebuf.at[1]],
                                      ssem).wait()

            issue(ebufA, gidxA, wbufA, rowsA, semA, 0)
            issue(ebufB, gidxB, wbufB, rowsB, semB, 1)

            def pair(kk, carry):
                i0 = kk * 2
                process(ebufA, gidxA, wbufA, rowsA, semA)
                scat(ebufA, rowsA, ssemA)
                process(ebufB, gidxB, wbufB, rowsB, semB)
                scat(ebufB, rowsB, ssemB)

                @pl.when(kk < NCB // 2 - 1)
                def _():
                    drain(ebufA, rowsA, ssemA)
                    issue(ebufA, gidxA, wbufA, rowsA, semA, i0 + 2)
                    drain(ebufB, rowsB, ssemB)
                    issue(ebufB, gidxB, wbufB, rowsB, semB, i0 + 3)

                return carry

            lax.fori_loop(0, NCB // 2, pair, 0)
            drain(ebufA, rowsA, ssemA)
            drain(ebufB, rowsB, ssemB)
            plsc.subcore_barrier()
            pltpu.sync_copy(
                numsh.at[pl.ds(row0, ROWS_PT)],
                out_hbm.at[pl.ds(qN + row0, ROWS_PT)],
            )

    return kb


def _ones_cols(n):
    col = lax.broadcasted_iota(jnp.int32, (n, 16), 1)
    return jnp.where(col < 2, 1.0, 0.0).astype(jnp.float32)


def _write_vaug(v_ref, v):
    ones = _ones_cols(v.shape[0])
    nchunk = v_ref.shape[0]
    cw = v_ref.shape[2] - 16
    for qc in range(nchunk):
        v_ref[qc] = jnp.concatenate([v[:, qc * cw:(qc + 1) * cw], ones],
                                    axis=1)


def _tc1_body(x_ref, wq, bq, wk, bk, wv, bv, ws, bs, q_ref, k_ref, s_ref,
              v_ref):
    xb = x_ref[...]
    q_ref[...] = jnp.dot(xb, wq[...], preferred_element_type=jnp.float32) + bq[...]
    k_ref[...] = jnp.dot(xb, wk[...], preferred_element_type=jnp.float32) + bk[...]
    s_ref[...] = jnp.dot(xb, ws[...], preferred_element_type=jnp.float32) + bs[...]
    v = jnp.dot(xb, wv[...], preferred_element_type=jnp.float32) + bv[...]
    _write_vaug(v_ref, v)


def _norm_heads(na, C):
    cols = []
    for qc in range(na.shape[0]):
        seg = na[qc]
        d0 = seg[:, 2 * C:2 * C + 1] + EPS
        d1 = seg[:, 2 * C + 1:2 * C + 2] + EPS
        cols.append(seg[:, 0:C] / d0)
        cols.append(seg[:, C:2 * C] / d1)
    return jnp.concatenate(cols, axis=1)


def _tc2_body(na_ref, s1_ref, wq, bq, wk, bk, wv, bv, ws, bs,
              q_ref, k_ref, s_ref, v_ref):
    h = _norm_heads(na_ref[...], 32) + s1_ref[...]
    h = jnp.maximum(h, 0.0)
    q = jnp.dot(h, wq[...], preferred_element_type=jnp.float32) + bq[...]
    k = jnp.dot(h, wk[...], preferred_element_type=jnp.float32) + bk[...]
    q_ref[0] = q[:, :256]
    q_ref[1] = q[:, 256:]
    k_ref[0] = k[:, :256]
    k_ref[1] = k[:, 256:]
    s_ref[...] = jnp.dot(h, ws[...], preferred_element_type=jnp.float32) + bs[...]
    v = jnp.dot(h, wv[...], preferred_element_type=jnp.float32) + bv[...]
    _write_vaug(v_ref, v)


def _tc3_body(na_ref, s2_ref, out_ref):
    out_ref[...] = _norm_heads(na_ref[...], 64) + s2_ref[...]


def _full(shape):
    return pl.BlockSpec(shape, lambda i: tuple(0 for _ in shape))


def _rows(shape):
    return pl.BlockSpec(shape, lambda i: (i,) + tuple(0 for _ in shape[1:]))


def _mid(shape):
    return pl.BlockSpec(shape, lambda i: (0, i) + tuple(0 for _ in shape[2:]))


def _tc1(x, wq, bq, wk, bk, wv, bv, ws, bs):
    return pl.pallas_call(
        _tc1_body,
        grid=(N // BN,),
        in_specs=[_rows((BN, 16))] + [
            _full(w.shape) for w in (wq, bq, wk, bk, wv, bv, ws, bs)],
        out_specs=[_rows((BN, 128)), _rows((BN, 128)), _rows((BN, 128)),
                   _mid((2, BN, 80))],
        out_shape=[jax.ShapeDtypeStruct((N, 128), jnp.float32)] * 3 +
                  [jax.ShapeDtypeStruct((2, N, 80), jnp.float32)],
    )(x, wq, bq, wk, bk, wv, bv, ws, bs)


def _tc2(na1, s1, wq, bq, wk, bk, wv, bv, ws, bs):
    return pl.pallas_call(
        _tc2_body,
        grid=(N // BN,),
        in_specs=[_mid((2, BN, 80)), _rows((BN, 128))] + [
            _full(w.shape) for w in (wq, bq, wk, bk, wv, bv, ws, bs)],
        out_specs=[_mid((2, BN, 256)), _mid((2, BN, 256)), _rows((BN, 512)),
                   _mid((4, BN, 144))],
        out_shape=[jax.ShapeDtypeStruct((2, N, 256), jnp.float32)] * 2 +
                  [jax.ShapeDtypeStruct((N, 512), jnp.float32),
                   jax.ShapeDtypeStruct((4, N, 144), jnp.float32)],
    )(na1, s1, wq, bq, wk, bk, wv, bv, ws, bs)


def _tc3(na2, s2):
    return pl.pallas_call(
        _tc3_body,
        grid=(N // BN,),
        in_specs=[_mid((4, BN, 144)), _rows((BN, 512))],
        out_specs=_rows((BN, 512)),
        out_shape=jax.ShapeDtypeStruct((N, 512), jnp.float32),
    )(na2, s2)


def kernel(x, edge_index, Wq1, bq1, Wk1, bk1, Wv1, bv1, Ws1, bs1,
           Wq2, bq2, Wk2, bk2, Wv2, bv2, Ws2, bs2):
    eip = jnp.pad(edge_index.astype(jnp.int32), ((0, 0), (0, EPAD - E)))

    q1, k1, s1, vaug1 = _tc1(x, Wq1, bq1, Wk1, bk1, Wv1, bv1, Ws1, bs1)
    wt1 = _make_phase_a(128, 4, 32, 1)(q1, k1, eip)
    na1 = _make_phase_b(128, 4, 32, 2)(vaug1.reshape(2 * N, 80), eip, wt1)
    qs, ks, s2, vaug2 = _tc2(na1.reshape(2, N, 80), s1,
                             Wq2, bq2, Wk2, bk2, Wv2, bv2, Ws2, bs2)
    wt2 = _make_phase_a(512, 8, 64, 2)(qs[0], qs[1], ks[0], ks[1], eip)
    na2 = _make_phase_b(512, 8, 64, 4)(vaug2.reshape(4 * N, 144), eip, wt2)
    return _tc3(na2.reshape(4, N, 144), s2)


# speedup vs baseline: 1.0220x; 1.0220x over previous
import functools
import math

import jax
import jax.numpy as jnp
from jax import lax
from jax.experimental import pallas as pl
from jax.experimental.pallas import tpu as pltpu
from jax.experimental.pallas import tpu_sc as plsc

N = 10000
E = 160000
EPAD = 163840
NCORE = 2
NSUB = 16
NW = NCORE * NSUB
BN = 1000
EPS = 1e-16

_SC_PARAMS = pltpu.CompilerParams(
    use_tc_tiling_on_sc=False, needs_layout_passes=False)


def _mesh():
    return plsc.VectorSubcoreMesh(core_axis_name="c", subcore_axis_name="s")


def _make_phase_a(D, H, C, HG):
    DH = D // HG
    HH = H // HG
    EPT = EPAD // NW
    BA = 64
    NCH = EPT // BA
    inv = 1.0 / math.sqrt(C)

    @functools.partial(
        pl.kernel,
        mesh=_mesh(),
        out_type=jax.ShapeDtypeStruct((H, EPAD), jnp.float32),
        scratch_types=[
            pltpu.VMEM((2, BA), jnp.int32),
            pltpu.VMEM((2, BA), jnp.int32),
            pltpu.VMEM((BA, DH), jnp.float32),
            pltpu.VMEM((BA, DH), jnp.float32),
            pltpu.VMEM((BA, DH), jnp.float32),
            pltpu.VMEM((BA, DH), jnp.float32),
            pltpu.VMEM((HH, BA), jnp.float32),
            pltpu.SemaphoreType.DMA,
            pltpu.SemaphoreType.DMA,
        ],
        compiler_params=_SC_PARAMS,
        name=f"edge_attn_w_D{D}",
    )
    def ka(*refs):
        q_tabs = refs[0:HG]
        k_tabs = refs[HG:2 * HG]
        ei_hbm = refs[2 * HG]
        wt_hbm = refs[2 * HG + 1]
        (ebufA, ebufB, qrA, qrB, krA, krB, wbuf, semA, semB) = refs[2 * HG + 2:]
        cid = lax.axis_index("c")
        sid = lax.axis_index("s")
        wid = cid * NSUB + sid
        base = wid * EPT
        lane = lax.broadcasted_iota(jnp.int32, (16,), 0)

        def start_gathers(q_hbm, k_hbm, ebuf, qr, kr, sem, off):
            pltpu.sync_copy(ei_hbm.at[:, pl.ds(off, BA)], ebuf)
            ck = pltpu.async_copy(k_hbm.at[ebuf.at[0]], kr, sem)
            cq = pltpu.async_copy(q_hbm.at[ebuf.at[1]], qr, sem)
            return ck, cq

        for p in range(HG):
            q_hbm = q_tabs[p]
            k_hbm = k_tabs[p]

            def issue(ebuf, qr, kr, sem, ci):
                off = pl.multiple_of(base + ci * BA, 8)
                return start_gathers(q_hbm, k_hbm, ebuf, qr, kr, sem, off)

            perms = [lane ^ s for s in (1, 2, 4, 8)]

            def hsum(v):
                for pm in perms:
                    v = v + v.at[pm].get(mode="promise_in_bounds")
                return v

            def process(ebuf, qr, kr, sem, ci):
                pltpu.make_async_copy(k_hbm.at[ebuf.at[0]], kr, sem).wait()
                pltpu.make_async_copy(q_hbm.at[ebuf.at[1]], qr, sem).wait()
                off = pl.multiple_of(base + ci * BA, 8)

                def group(g, carry):
                    valid = (off + g * 16 + lane) < E
                    wvecs = [jnp.zeros((16,), jnp.float32)] * HH
                    for j in range(16):
                        e = g * 16 + j
                        for h in range(HH):
                            acc = None
                            for t in range(C // 16):
                                sl = pl.ds(h * C + t * 16, 16)
                                prod = qr[e, sl] * kr[e, sl]
                                acc = prod if acc is None else acc + prod
                            s = hsum(acc)
                            wvecs[h] = jnp.where(lane == j, s, wvecs[h])
                    for h in range(HH):
                        w = jnp.exp(wvecs[h] * inv)
                        w = jnp.where(valid, w, 0.0)
                        wbuf[h, pl.ds(g * 16, 16)] = w
                    return carry

                lax.fori_loop(0, BA // 16, group, 0)
                pltpu.sync_copy(
                    wbuf,
                    wt_hbm.at[pl.ds(p * HH, HH), pl.ds(off, BA)])

            issue(ebufA, qrA, krA, semA, 0)
            issue(ebufB, qrB, krB, semB, 1)

            def pair(kk, carry):
                i0 = kk * 2
                process(ebufA, qrA, krA, semA, i0)

                @pl.when(kk < NCH // 2 - 1)
                def _():
                    issue(ebufA, qrA, krA, semA, i0 + 2)

                process(ebufB, qrB, krB, semB, i0 + 1)

                @pl.when(kk < NCH // 2 - 1)
                def _():
                    issue(ebufB, qrB, krB, semB, i0 + 3)

                return carry

            lax.fori_loop(0, NCH // 2, pair, 0)

    return ka


def _make_phase_b(D, H, C, NCHUNK):
    CW = 2 * C
    CWA = CW + 16
    JH = C // 16
    NPASS = NCHUNK // NCORE
    EPT2 = EPAD // NSUB
    BB = 128
    NCB = EPT2 // BB
    ROWS_PT = N // NSUB
    ZR = 25

    @functools.partial(
        pl.kernel,
        mesh=_mesh(),
        out_type=jax.ShapeDtypeStruct((NCHUNK * N, CWA), jnp.float32),
        scratch_types=[
            pltpu.VMEM((2, BB), jnp.int32),
            pltpu.VMEM((2, BB), jnp.int32),
            pltpu.VMEM((BB,), jnp.int32),
            pltpu.VMEM((BB,), jnp.int32),
            pltpu.VMEM((2, BB), jnp.float32),
            pltpu.VMEM((2, BB), jnp.float32),
            pltpu.VMEM((BB, CWA), jnp.float32),
            pltpu.VMEM((BB, CWA), jnp.float32),
            pltpu.VMEM_SHARED((N, CWA), jnp.float32),
            pltpu.SemaphoreType.DMA,
            pltpu.SemaphoreType.DMA,
        ],
        compiler_params=_SC_PARAMS,
        name=f"edge_scatter_D{D}",
    )
    def kb(vaug_hbm, ei_hbm, wt_hbm, out_hbm, ebufA, ebufB, gidxA, gidxB,
           wbufA, wbufB, rowsA, rowsB, numsh, semA, semB):
        cid = lax.axis_index("c")
        sid = lax.axis_index("s")
        lane = lax.broadcasted_iota(jnp.int32, (16,), 0)
        m0 = jnp.where(lane == 0, 1.0, 0.0)
        m1 = jnp.where(lane == 1, 1.0, 0.0)
        zero16 = jnp.zeros((16,), jnp.float32)
        row0 = sid * ROWS_PT

        for p in range(NPASS):
            q = p * NCORE + cid
            qN = q * N
            h0 = q * 2
            for zr in range(ZR):
                for j in range(CWA // 16):
                    rowsA[zr, pl.ds(j * 16, 16)] = zero16
            for zi in range(ROWS_PT // ZR):
                pltpu.sync_copy(rowsA.at[pl.ds(0, ZR)],
                                numsh.at[pl.ds(row0 + zi * ZR, ZR)])
            plsc.subcore_barrier()

            def issue(ebuf, gidx, wbuf, rows, sem, ci):
                off = pl.multiple_of(sid * EPT2 + ci * BB, 8)
                pltpu.sync_copy(ei_hbm.at[:, pl.ds(off, BB)], ebuf)
                pltpu.sync_copy(
                    wt_hbm.at[pl.ds(h0, 2), pl.ds(off, BB)], wbuf)
                for j in range(BB // 16):
                    sl = pl.ds(j * 16, 16)
                    gidx[sl] = ebuf[0, sl] + qN
                pltpu.async_copy(vaug_hbm.at[gidx], rows, sem)

            def process(ebuf, gidx, wbuf, rows, sem):
                pltpu.make_async_copy(vaug_hbm.at[gidx], rows, sem).wait()

                def group(g, c2):
                    w0vec = wbuf[0, pl.ds(g * 16, 16)]
                    w1vec = wbuf[1, pl.ds(g * 16, 16)]
                    for j in range(16):
                        e = g * 16 + j
                        jsplat = jnp.full((16,), j, jnp.int32)
                        w0 = w0vec.at[jsplat].get(mode="promise_in_bounds")
                        w1 = w1vec.at[jsplat].get(mode="promise_in_bounds")
                        s8 = w0 * m0 + w1 * m1
                        for t in range(JH):
                            sl = pl.ds(t * 16, 16)
                            rows[e, sl] = rows[e, sl] * w0
                        for t in range(JH):
                            sl = pl.ds(C + t * 16, 16)
                            rows[e, sl] = rows[e, sl] * w1
                        sl = pl.ds(CW, 16)
                        rows[e, sl] = rows[e, sl] * s8
                    return c2

                lax.fori_loop(0, BB // 16, group, 0)
                pltpu.sync_copy(rows, numsh.at[ebuf.at[1]], add=True)

            issue(ebufA, gidxA, wbufA, rowsA, semA, 0)
            issue(ebufB, gidxB, wbufB, rowsB, semB, 1)

            def pair(kk, carry):
                i0 = kk * 2
                process(ebufA, gidxA, wbufA, rowsA, semA)

                @pl.when(kk < NCB // 2 - 1)
                def _():
                    issue(ebufA, gidxA, wbufA, rowsA, semA, i0 + 2)

                process(ebufB, gidxB, wbufB, rowsB, semB)

                @pl.when(kk < NCB // 2 - 1)
                def _():
                    issue(ebufB, gidxB, wbufB, rowsB, semB, i0 + 3)

                return carry

            lax.fori_loop(0, NCB // 2, pair, 0)
            plsc.subcore_barrier()
            pltpu.sync_copy(
                numsh.at[pl.ds(row0, ROWS_PT)],
                out_hbm.at[pl.ds(qN + row0, ROWS_PT)],
            )

    return kb


def _ones_cols(n):
    col = lax.broadcasted_iota(jnp.int32, (n, 16), 1)
    return jnp.where(col < 2, 1.0, 0.0).astype(jnp.float32)


def _write_vaug(v_ref, v):
    ones = _ones_cols(v.shape[0])
    nchunk = v_ref.shape[0]
    cw = v_ref.shape[2] - 16
    for qc in range(nchunk):
        v_ref[qc] = jnp.concatenate([v[:, qc * cw:(qc + 1) * cw], ones],
                                    axis=1)


def _tc1_body(x_ref, wq, bq, wk, bk, wv, bv, ws, bs, q_ref, k_ref, s_ref,
              v_ref):
    xb = x_ref[...]
    q_ref[...] = jnp.dot(xb, wq[...], preferred_element_type=jnp.float32) + bq[...]
    k_ref[...] = jnp.dot(xb, wk[...], preferred_element_type=jnp.float32) + bk[...]
    s_ref[...] = jnp.dot(xb, ws[...], preferred_element_type=jnp.float32) + bs[...]
    v = jnp.dot(xb, wv[...], preferred_element_type=jnp.float32) + bv[...]
    _write_vaug(v_ref, v)


def _norm_heads(na, C):
    cols = []
    for qc in range(na.shape[0]):
        seg = na[qc]
        d0 = seg[:, 2 * C:2 * C + 1] + EPS
        d1 = seg[:, 2 * C + 1:2 * C + 2] + EPS
        cols.append(seg[:, 0:C] / d0)
        cols.append(seg[:, C:2 * C] / d1)
    return jnp.concatenate(cols, axis=1)


def _tc2_body(na_ref, s1_ref, wq, bq, wk, bk, wv, bv, ws, bs,
              q_ref, k_ref, s_ref, v_ref):
    h = _norm_heads(na_ref[...], 32) + s1_ref[...]
    h = jnp.maximum(h, 0.0)
    q = jnp.dot(h, wq[...], preferred_element_type=jnp.float32) + bq[...]
    k = jnp.dot(h, wk[...], preferred_element_type=jnp.float32) + bk[...]
    q_ref[0] = q[:, :256]
    q_ref[1] = q[:, 256:]
    k_ref[0] = k[:, :256]
    k_ref[1] = k[:, 256:]
    s_ref[...] = jnp.dot(h, ws[...], preferred_element_type=jnp.float32) + bs[...]
    v = jnp.dot(h, wv[...], preferred_element_type=jnp.float32) + bv[...]
    _write_vaug(v_ref, v)


def _tc3_body(na_ref, s2_ref, out_ref):
    out_ref[...] = _norm_heads(na_ref[...], 64) + s2_ref[...]


def _full(shape):
    return pl.BlockSpec(shape, lambda i: tuple(0 for _ in shape))


def _rows(shape):
    return pl.BlockSpec(shape, lambda i: (i,) + tuple(0 for _ in shape[1:]))


def _mid(shape):
    return pl.BlockSpec(shape, lambda i: (0, i) + tuple(0 for _ in shape[2:]))


def _tc1(x, wq, bq, wk, bk, wv, bv, ws, bs):
    return pl.pallas_call(
        _tc1_body,
        grid=(N // BN,),
        in_specs=[_rows((BN, 16))] + [
            _full(w.shape) for w in (wq, bq, wk, bk, wv, bv, ws, bs)],
        out_specs=[_rows((BN, 128)), _rows((BN, 128)), _rows((BN, 128)),
                   _mid((2, BN, 80))],
        out_shape=[jax.ShapeDtypeStruct((N, 128), jnp.float32)] * 3 +
                  [jax.ShapeDtypeStruct((2, N, 80), jnp.float32)],
    )(x, wq, bq, wk, bk, wv, bv, ws, bs)


def _tc2(na1, s1, wq, bq, wk, bk, wv, bv, ws, bs):
    return pl.pallas_call(
        _tc2_body,
        grid=(N // BN,),
        in_specs=[_mid((2, BN, 80)), _rows((BN, 128))] + [
            _full(w.shape) for w in (wq, bq, wk, bk, wv, bv, ws, bs)],
        out_specs=[_mid((2, BN, 256)), _mid((2, BN, 256)), _rows((BN, 512)),
                   _mid((4, BN, 144))],
        out_shape=[jax.ShapeDtypeStruct((2, N, 256), jnp.float32)] * 2 +
                  [jax.ShapeDtypeStruct((N, 512), jnp.float32),
                   jax.ShapeDtypeStruct((4, N, 144), jnp.float32)],
    )(na1, s1, wq, bq, wk, bk, wv, bv, ws, bs)


def _tc3(na2, s2):
    return pl.pallas_call(
        _tc3_body,
        grid=(N // BN,),
        in_specs=[_mid((4, BN, 144)), _rows((BN, 512))],
        out_specs=_rows((BN, 512)),
        out_shape=jax.ShapeDtypeStruct((N, 512), jnp.float32),
    )(na2, s2)


def kernel(x, edge_index, Wq1, bq1, Wk1, bk1, Wv1, bv1, Ws1, bs1,
           Wq2, bq2, Wk2, bk2, Wv2, bv2, Ws2, bs2):
    eip = jnp.pad(edge_index.astype(jnp.int32), ((0, 0), (0, EPAD - E)))

    q1, k1, s1, vaug1 = _tc1(x, Wq1, bq1, Wk1, bk1, Wv1, bv1, Ws1, bs1)
    wt1 = _make_phase_a(128, 4, 32, 1)(q1, k1, eip)
    na1 = _make_phase_b(128, 4, 32, 2)(vaug1.reshape(2 * N, 80), eip, wt1)
    qs, ks, s2, vaug2 = _tc2(na1.reshape(2, N, 80), s1,
                             Wq2, bq2, Wk2, bk2, Wv2, bv2, Ws2, bs2)
    wt2 = _make_phase_a(512, 8, 64, 2)(qs[0], qs[1], ks[0], ks[1], eip)
    na2 = _make_phase_b(512, 8, 64, 4)(vaug2.reshape(4 * N, 144), eip, wt2)
    return _tc3(na2.reshape(4, N, 144), s2)


# speedup vs baseline: 1.0361x; 1.0138x over previous
import functools
import math

import jax
import jax.numpy as jnp
from jax import lax
from jax.experimental import pallas as pl
from jax.experimental.pallas import tpu as pltpu
from jax.experimental.pallas import tpu_sc as plsc

N = 10000
E = 160000
EPAD = 163840
NCORE = 2
NSUB = 16
NW = NCORE * NSUB
BN = 1000
EPS = 1e-16

_SC_PARAMS = pltpu.CompilerParams(
    use_tc_tiling_on_sc=False, needs_layout_passes=False)


def _mesh():
    return plsc.VectorSubcoreMesh(core_axis_name="c", subcore_axis_name="s")


def _make_phase_a(D, H, C, HG):
    DH = D // HG
    HH = H // HG
    EPT = EPAD // NW
    BA = 64
    NCH = EPT // BA
    inv = 1.0 / math.sqrt(C)

    @functools.partial(
        pl.kernel,
        mesh=_mesh(),
        out_type=jax.ShapeDtypeStruct((H, EPAD), jnp.float32),
        scratch_types=[
            pltpu.VMEM((2, EPT), jnp.int32),
            pltpu.VMEM((BA, DH), jnp.float32),
            pltpu.VMEM((BA, DH), jnp.float32),
            pltpu.VMEM((BA, DH), jnp.float32),
            pltpu.VMEM((BA, DH), jnp.float32),
            pltpu.VMEM((HH, BA), jnp.float32),
            pltpu.SemaphoreType.DMA,
            pltpu.SemaphoreType.DMA,
        ],
        compiler_params=_SC_PARAMS,
        name=f"edge_attn_w_D{D}",
    )
    def ka(*refs):
        q_tabs = refs[0:HG]
        k_tabs = refs[HG:2 * HG]
        ei_hbm = refs[2 * HG]
        wt_hbm = refs[2 * HG + 1]
        (ebig, qrA, qrB, krA, krB, wbuf, semA, semB) = refs[2 * HG + 2:]
        cid = lax.axis_index("c")
        sid = lax.axis_index("s")
        wid = cid * NSUB + sid
        base = wid * EPT
        lane = lax.broadcasted_iota(jnp.int32, (16,), 0)
        pltpu.sync_copy(ei_hbm.at[:, pl.ds(base, EPT)], ebig)

        for p in range(HG):
            q_hbm = q_tabs[p]
            k_hbm = k_tabs[p]

            def issue(qr, kr, sem, ci):
                loff = pl.multiple_of(ci * BA, 8)
                pltpu.async_copy(
                    k_hbm.at[ebig.at[0, pl.ds(loff, BA)]], kr, sem)
                pltpu.async_copy(
                    q_hbm.at[ebig.at[1, pl.ds(loff, BA)]], qr, sem)

            perms = [lane ^ s for s in (1, 2, 4, 8)]

            def hsum(v):
                for pm in perms:
                    v = v + v.at[pm].get(mode="promise_in_bounds")
                return v

            def process(qr, kr, sem, ci):
                loff = pl.multiple_of(ci * BA, 8)
                pltpu.make_async_copy(
                    k_hbm.at[ebig.at[0, pl.ds(loff, BA)]], kr, sem).wait()
                pltpu.make_async_copy(
                    q_hbm.at[ebig.at[1, pl.ds(loff, BA)]], qr, sem).wait()
                off = pl.multiple_of(base + ci * BA, 8)

                def group(g, carry):
                    valid = (off + g * 16 + lane) < E
                    wvecs = [jnp.zeros((16,), jnp.float32)] * HH
                    for j in range(16):
                        e = g * 16 + j
                        for h in range(HH):
                            acc = None
                            for t in range(C // 16):
                                sl = pl.ds(h * C + t * 16, 16)
                                prod = qr[e, sl] * kr[e, sl]
                                acc = prod if acc is None else acc + prod
                            s = hsum(acc)
                            wvecs[h] = jnp.where(lane == j, s, wvecs[h])
                    for h in range(HH):
                        w = jnp.exp(wvecs[h] * inv)
                        w = jnp.where(valid, w, 0.0)
                        wbuf[h, pl.ds(g * 16, 16)] = w
                    return carry

                lax.fori_loop(0, BA // 16, group, 0)
                pltpu.sync_copy(
                    wbuf,
                    wt_hbm.at[pl.ds(p * HH, HH), pl.ds(off, BA)])

            issue(qrA, krA, semA, 0)
            issue(qrB, krB, semB, 1)

            def pair(kk, carry):
                i0 = kk * 2
                process(qrA, krA, semA, i0)

                @pl.when(kk < NCH // 2 - 1)
                def _():
                    issue(qrA, krA, semA, i0 + 2)

                process(qrB, krB, semB, i0 + 1)

                @pl.when(kk < NCH // 2 - 1)
                def _():
                    issue(qrB, krB, semB, i0 + 3)

                return carry

            lax.fori_loop(0, NCH // 2, pair, 0)

    return ka


def _make_phase_b(D, H, C, NCHUNK):
    CW = 2 * C
    CWA = CW + 16
    JH = C // 16
    NPASS = NCHUNK // NCORE
    EPT2 = EPAD // NSUB
    BB = 128
    NCB = EPT2 // BB
    ROWS_PT = N // NSUB
    ZR = 25

    @functools.partial(
        pl.kernel,
        mesh=_mesh(),
        out_type=jax.ShapeDtypeStruct((NCHUNK * N, CWA), jnp.float32),
        scratch_types=[
            pltpu.VMEM((2, BB), jnp.int32),
            pltpu.VMEM((2, BB), jnp.int32),
            pltpu.VMEM((BB,), jnp.int32),
            pltpu.VMEM((BB,), jnp.int32),
            pltpu.VMEM((2, BB), jnp.float32),
            pltpu.VMEM((2, BB), jnp.float32),
            pltpu.VMEM((BB, CWA), jnp.float32),
            pltpu.VMEM((BB, CWA), jnp.float32),
            pltpu.VMEM_SHARED((N, CWA), jnp.float32),
            pltpu.SemaphoreType.DMA,
            pltpu.SemaphoreType.DMA,
        ],
        compiler_params=_SC_PARAMS,
        name=f"edge_scatter_D{D}",
    )
    def kb(vaug_hbm, ei_hbm, wt_hbm, out_hbm, ebufA, ebufB, gidxA, gidxB,
           wbufA, wbufB, rowsA, rowsB, numsh, semA, semB):
        cid = lax.axis_index("c")
        sid = lax.axis_index("s")
        lane = lax.broadcasted_iota(jnp.int32, (16,), 0)
        m0 = jnp.where(lane == 0, 1.0, 0.0)
        m1 = jnp.where(lane == 1, 1.0, 0.0)
        zero16 = jnp.zeros((16,), jnp.float32)
        row0 = sid * ROWS_PT

        for p in range(NPASS):
            q = p * NCORE + cid
            qN = q * N
            h0 = q * 2
            for zr in range(ZR):
                for j in range(CWA // 16):
                    rowsA[zr, pl.ds(j * 16, 16)] = zero16
            for zi in range(ROWS_PT // ZR):
                pltpu.sync_copy(rowsA.at[pl.ds(0, ZR)],
                                numsh.at[pl.ds(row0 + zi * ZR, ZR)])
            plsc.subcore_barrier()

            def issue(ebuf, gidx, wbuf, rows, sem, ci):
                off = pl.multiple_of(sid * EPT2 + ci * BB, 8)
                pltpu.sync_copy(ei_hbm.at[:, pl.ds(off, BB)], ebuf)
                pltpu.sync_copy(
                    wt_hbm.at[pl.ds(h0, 2), pl.ds(off, BB)], wbuf)
                for j in range(BB // 16):
                    sl = pl.ds(j * 16, 16)
                    gidx[sl] = ebuf[0, sl] + qN
                pltpu.async_copy(vaug_hbm.at[gidx], rows, sem)

            def process(ebuf, gidx, wbuf, rows, sem):
                pltpu.make_async_copy(vaug_hbm.at[gidx], rows, sem).wait()

                def group(g, c2):
                    w0vec = wbuf[0, pl.ds(g * 16, 16)]
                    w1vec = wbuf[1, pl.ds(g * 16, 16)]
                    for j in range(16):
                        e = g * 16 + j
                        jsplat = jnp.full((16,), j, jnp.int32)
                        w0 = w0vec.at[jsplat].get(mode="promise_in_bounds")
                        w1 = w1vec.at[jsplat].get(mode="promise_in_bounds")
                        s8 = w0 * m0 + w1 * m1
                        for t in range(JH):
                            sl = pl.ds(t * 16, 16)
                            rows[e, sl] = rows[e, sl] * w0
                        for t in range(JH):
                            sl = pl.ds(C + t * 16, 16)
                            rows[e, sl] = rows[e, sl] * w1
                        sl = pl.ds(CW, 16)
                        rows[e, sl] = rows[e, sl] * s8
                    return c2

                lax.fori_loop(0, BB // 16, group, 0)
                pltpu.sync_copy(rows, numsh.at[ebuf.at[1]], add=True)

            issue(ebufA, gidxA, wbufA, rowsA, semA, 0)
            issue(ebufB, gidxB, wbufB, rowsB, semB, 1)

            def pair(kk, carry):
                i0 = kk * 2
                process(ebufA, gidxA, wbufA, rowsA, semA)

                @pl.when(kk < NCB // 2 - 1)
                def _():
                    issue(ebufA, gidxA, wbufA, rowsA, semA, i0 + 2)

                process(ebufB, gidxB, wbufB, rowsB, semB)

                @pl.when(kk < NCB // 2 - 1)
                def _():
                    issue(ebufB, gidxB, wbufB, rowsB, semB, i0 + 3)

                return carry

            lax.fori_loop(0, NCB // 2, pair, 0)
            plsc.subcore_barrier()
            pltpu.sync_copy(
                numsh.at[pl.ds(row0, ROWS_PT)],
                out_hbm.at[pl.ds(qN + row0, ROWS_PT)],
            )

    return kb


def _ones_cols(n):
    col = lax.broadcasted_iota(jnp.int32, (n, 16), 1)
    return jnp.where(col < 2, 1.0, 0.0).astype(jnp.float32)


def _write_vaug(v_ref, v):
    ones = _ones_cols(v.shape[0])
    nchunk = v_ref.shape[0]
    cw = v_ref.shape[2] - 16
    for qc in range(nchunk):
        v_ref[qc] = jnp.concatenate([v[:, qc * cw:(qc + 1) * cw], ones],
                                    axis=1)


def _tc1_body(x_ref, wq, bq, wk, bk, wv, bv, ws, bs, q_ref, k_ref, s_ref,
              v_ref):
    xb = x_ref[...]
    q_ref[...] = jnp.dot(xb, wq[...], preferred_element_type=jnp.float32) + bq[...]
    k_ref[...] = jnp.dot(xb, wk[...], preferred_element_type=jnp.float32) + bk[...]
    s_ref[...] = jnp.dot(xb, ws[...], preferred_element_type=jnp.float32) + bs[...]
    v = jnp.dot(xb, wv[...], preferred_element_type=jnp.float32) + bv[...]
    _write_vaug(v_ref, v)


def _norm_heads(na, C):
    cols = []
    for qc in range(na.shape[0]):
        seg = na[qc]
        d0 = seg[:, 2 * C:2 * C + 1] + EPS
        d1 = seg[:, 2 * C + 1:2 * C + 2] + EPS
        cols.append(seg[:, 0:C] / d0)
        cols.append(seg[:, C:2 * C] / d1)
    return jnp.concatenate(cols, axis=1)


def _tc2_body(na_ref, s1_ref, wq, bq, wk, bk, wv, bv, ws, bs,
              q_ref, k_ref, s_ref, v_ref):
    h = _norm_heads(na_ref[...], 32) + s1_ref[...]
    h = jnp.maximum(h, 0.0)
    q = jnp.dot(h, wq[...], preferred_element_type=jnp.float32) + bq[...]
    k = jnp.dot(h, wk[...], preferred_element_type=jnp.float32) + bk[...]
    q_ref[0] = q[:, :256]
    q_ref[1] = q[:, 256:]
    k_ref[0] = k[:, :256]
    k_ref[1] = k[:, 256:]
    s_ref[...] = jnp.dot(h, ws[...], preferred_element_type=jnp.float32) + bs[...]
    v = jnp.dot(h, wv[...], preferred_element_type=jnp.float32) + bv[...]
    _write_vaug(v_ref, v)


def _tc3_body(na_ref, s2_ref, out_ref):
    out_ref[...] = _norm_heads(na_ref[...], 64) + s2_ref[...]


def _full(shape):
    return pl.BlockSpec(shape, lambda i: tuple(0 for _ in shape))


def _rows(shape):
    return pl.BlockSpec(shape, lambda i: (i,) + tuple(0 for _ in shape[1:]))


def _mid(shape):
    return pl.BlockSpec(shape, lambda i: (0, i) + tuple(0 for _ in shape[2:]))


def _tc1(x, wq, bq, wk, bk, wv, bv, ws, bs):
    return pl.pallas_call(
        _tc1_body,
        grid=(N // BN,),
        in_specs=[_rows((BN, 16))] + [
            _full(w.shape) for w in (wq, bq, wk, bk, wv, bv, ws, bs)],
        out_specs=[_rows((BN, 128)), _rows((BN, 128)), _rows((BN, 128)),
                   _mid((2, BN, 80))],
        out_shape=[jax.ShapeDtypeStruct((N, 128), jnp.float32)] * 3 +
                  [jax.ShapeDtypeStruct((2, N, 80), jnp.float32)],
    )(x, wq, bq, wk, bk, wv, bv, ws, bs)


def _tc2(na1, s1, wq, bq, wk, bk, wv, bv, ws, bs):
    return pl.pallas_call(
        _tc2_body,
        grid=(N // BN,),
        in_specs=[_mid((2, BN, 80)), _rows((BN, 128))] + [
            _full(w.shape) for w in (wq, bq, wk, bk, wv, bv, ws, bs)],
        out_specs=[_mid((2, BN, 256)), _mid((2, BN, 256)), _rows((BN, 512)),
                   _mid((4, BN, 144))],
        out_shape=[jax.ShapeDtypeStruct((2, N, 256), jnp.float32)] * 2 +
                  [jax.ShapeDtypeStruct((N, 512), jnp.float32),
                   jax.ShapeDtypeStruct((4, N, 144), jnp.float32)],
    )(na1, s1, wq, bq, wk, bk, wv, bv, ws, bs)


def _tc3(na2, s2):
    return pl.pallas_call(
        _tc3_body,
        grid=(N // BN,),
        in_specs=[_mid((4, BN, 144)), _rows((BN, 512))],
        out_specs=_rows((BN, 512)),
        out_shape=jax.ShapeDtypeStruct((N, 512), jnp.float32),
    )(na2, s2)


def kernel(x, edge_index, Wq1, bq1, Wk1, bk1, Wv1, bv1, Ws1, bs1,
           Wq2, bq2, Wk2, bk2, Wv2, bv2, Ws2, bs2):
    eip = jnp.pad(edge_index.astype(jnp.int32), ((0, 0), (0, EPAD - E)))

    q1, k1, s1, vaug1 = _tc1(x, Wq1, bq1, Wk1, bk1, Wv1, bv1, Ws1, bs1)
    wt1 = _make_phase_a(128, 4, 32, 1)(q1, k1, eip)
    na1 = _make_phase_b(128, 4, 32, 2)(vaug1.reshape(2 * N, 80), eip, wt1)
    qs, ks, s2, vaug2 = _tc2(na1.reshape(2, N, 80), s1,
                             Wq2, bq2, Wk2, bk2, Wv2, bv2, Ws2, bs2)
    wt2 = _make_phase_a(512, 8, 64, 2)(qs[0], qs[1], ks[0], ks[1], eip)
    na2 = _make_phase_b(512, 8, 64, 4)(vaug2.reshape(4 * N, 144), eip, wt2)
    return _tc3(na2.reshape(4, N, 144), s2)
